# Initial kernel scaffold; baseline (speedup 1.0000x reference)
import jax
import jax.numpy as jnp
from jax.experimental import pallas as pl


def kernel(z, edge_index, s, Wi_reset, bi_reset, Ws_reset, bs_reset, Wi_update, bi_update, Ws_update, bs_update, Wi_cand, bi_cand, Ws_cand, bs_cand):
    raise NotImplementedError("write your pallas kernel here")


# speedup vs baseline: 17.5729x; 17.5729x over previous
import functools

import jax
import jax.numpy as jnp
from jax import lax
from jax.experimental import pallas as pl
from jax.experimental.pallas import tpu as pltpu
from jax.experimental.pallas import tpu_sc as plsc

N = 10000
E = 320000
D = 128
NC = 2
NS = 16
NPAD = 10240
ACC_ROWS = NPAD // NS
EPAD = 327680
IDXROWS = EPAD // 128
DEG_ROWS_PER_TILE = IDXROWS // (NC * NS)
KD = 8
ROWS_PER_TILE = IDXROWS // NS
K = 4

_MESH = plsc.VectorSubcoreMesh(core_axis_name="c", subcore_axis_name="s",
                               num_cores=NC, num_subcores=NS)


@functools.partial(
    pl.kernel,
    out_type=jax.ShapeDtypeStruct((NC * NPAD, 16), jnp.float32),
    mesh=_MESH,
    scratch_types=[
        pltpu.VMEM((KD, 128), jnp.int32),
        pltpu.VMEM((128, 16), jnp.float32),
        pltpu.VMEM((128, 16), jnp.float32),
        pltpu.VMEM_SHARED((NPAD, 16), jnp.float32),
    ],
)
def _deg_kernel(col_hbm, out_hbm, cidx_v, ones_v, zeros_v, acc_sh):
    c = lax.axis_index("c")
    s = lax.axis_index("s")

    def fill(i, _):
        ones_v[i] = jnp.ones((16,), jnp.float32)
        zeros_v[i] = jnp.zeros((16,), jnp.float32)
        return 0
    lax.fori_loop(0, 128, fill, 0)

    for q in range(ACC_ROWS // 128):
        pltpu.sync_copy(zeros_v, acc_sh.at[pl.ds(s * ACC_ROWS + q * 128, 128)])
    plsc.subcore_barrier()

    base = (c * NS + s) * DEG_ROWS_PER_TILE

    def chunk(it, _):
        pltpu.sync_copy(col_hbm.at[pl.ds(base + it * KD, KD)], cidx_v)
        for j in range(KD):
            pltpu.sync_copy(ones_v, acc_sh.at[cidx_v.at[j]], add=True)
        return 0
    lax.fori_loop(0, DEG_ROWS_PER_TILE // KD, chunk, 0)
    plsc.subcore_barrier()

    pltpu.sync_copy(acc_sh.at[pl.ds(s * ACC_ROWS, ACC_ROWS)],
                    out_hbm.at[pl.ds(c * NPAD + s * ACC_ROWS, ACC_ROWS)])


DH = D // 2


def _scale_body(z_ref, s_ref, degw_ref,
                tz0_ref, tz1_ref, ts0_ref, ts1_ref, dinv_ref):
    deg = degw_ref[0, :, 0:1] + degw_ref[1, :, 0:1] + 2.0
    dinv = lax.rsqrt(deg)
    tz = z_ref[...] * dinv
    ts = s_ref[...] * dinv
    tz0_ref[...] = tz[:, 0:DH]
    tz1_ref[...] = tz[:, DH:D]
    ts0_ref[...] = ts[:, 0:DH]
    ts1_ref[...] = ts[:, DH:D]
    dinv_ref[...] = jnp.broadcast_to(dinv, dinv_ref.shape)


_BN = 400


def _scale_call(z, s, degw):
    grid = N // _BN
    half = pl.BlockSpec((_BN, DH), lambda i: (i, 0))
    half_shape = jax.ShapeDtypeStruct((N, DH), jnp.float32)
    return pl.pallas_call(
        _scale_body,
        grid=(grid,),
        in_specs=[
            pl.BlockSpec((_BN, D), lambda i: (i, 0)),
            pl.BlockSpec((_BN, D), lambda i: (i, 0)),
            pl.BlockSpec((NC, _BN, 16), lambda i: (0, i, 0)),
        ],
        out_specs=[half, half, half, half,
                   pl.BlockSpec((_BN, 16), lambda i: (i, 0))],
        out_shape=[half_shape, half_shape, half_shape, half_shape,
                   jax.ShapeDtypeStruct((N, 16), jnp.float32)],
    )(z, s, degw)


@functools.partial(
    pl.kernel,
    out_type=jax.ShapeDtypeStruct((NC * 2 * NPAD, DH), jnp.float32),
    mesh=_MESH,
    compiler_params=pltpu.CompilerParams(use_tc_tiling_on_sc=False),
    scratch_types=[
        pltpu.VMEM((K, 128), jnp.int32),
        pltpu.VMEM((K, 128), jnp.int32),
        pltpu.VMEM((K * 128, DH), jnp.float32),
        pltpu.VMEM((128, DH), jnp.float32),
        pltpu.VMEM_SHARED((NPAD, DH), jnp.float32),
    ],
)
def _agg_kernel(tz0_hbm, tz1_hbm, ts0_hbm, ts1_hbm, row_hbm, col_hbm, out_hbm,
                ridx_v, cidx_v, gbuf_v, zeros_v, acc_sh):
    c = lax.axis_index("c")
    s = lax.axis_index("s")

    def fill(i, _):
        for j in range(DH // 16):
            zeros_v[i, pl.ds(j * 16, 16)] = jnp.zeros((16,), jnp.float32)
        return 0
    lax.fori_loop(0, 128, fill, 0)

    base = s * ROWS_PER_TILE

    for h, (taba, tabb) in enumerate(((tz0_hbm, ts0_hbm), (tz1_hbm, ts1_hbm))):
        for q in range(ACC_ROWS // 128):
            pltpu.sync_copy(zeros_v,
                            acc_sh.at[pl.ds(s * ACC_ROWS + q * 128, 128)])
        plsc.subcore_barrier()

        def chunk(it, _):
            pltpu.sync_copy(row_hbm.at[pl.ds(base + it * K, K)], ridx_v)
            pltpu.sync_copy(col_hbm.at[pl.ds(base + it * K, K)], cidx_v)

            @pl.when(c == 0)
            def _():
                for j in range(K):
                    pltpu.sync_copy(taba.at[ridx_v.at[j]],
                                    gbuf_v.at[pl.ds(j * 128, 128)])

            @pl.when(c == 1)
            def _():
                for j in range(K):
                    pltpu.sync_copy(tabb.at[ridx_v.at[j]],
                                    gbuf_v.at[pl.ds(j * 128, 128)])

            for j in range(K):
                pltpu.sync_copy(gbuf_v.at[pl.ds(j * 128, 128)],
                                acc_sh.at[cidx_v.at[j]], add=True)
            return 0
        lax.fori_loop(0, ROWS_PER_TILE // K, chunk, 0)
        plsc.subcore_barrier()

        pltpu.sync_copy(
            acc_sh.at[pl.ds(s * ACC_ROWS, ACC_ROWS)],
            out_hbm.at[pl.ds((c * 2 + h) * NPAD + s * ACC_ROWS, ACC_ROWS)])
        plsc.subcore_barrier()


def _gru_body(gz_ref, gs_ref, z_ref, s_ref, dinv_ref, wz_ref, ws_ref, b_ref,
              out_ref):
    d = dinv_ref[:, 0:1]
    two_d2 = 2.0 * d * d
    xz = d * gz_ref[0] + two_d2 * z_ref[...]
    xs = d * gs_ref[0] + two_d2 * s_ref[...]
    p = (jnp.dot(xz, wz_ref[...], preferred_element_type=jnp.float32,
                 precision=lax.Precision.HIGHEST)
         + jnp.dot(xs, ws_ref[...], preferred_element_type=jnp.float32,
                   precision=lax.Precision.HIGHEST)
         + b_ref[0:1, :])
    r = jax.nn.sigmoid(p[:, 0:D])
    u = jax.nn.sigmoid(p[:, D:2 * D])
    cand = jnp.tanh(p[:, 2 * D:3 * D] + r * p[:, 3 * D:4 * D])
    out_ref[...] = (1.0 - u) * cand + u * s_ref[...]


def _gru_call(agg, z, s, dinv16, wz, ws, bcat):
    grid = N // _BN
    return pl.pallas_call(
        _gru_body,
        grid=(grid,),
        in_specs=[
            pl.BlockSpec((1, _BN, D), lambda i: (0, i, 0)),
            pl.BlockSpec((1, _BN, D), lambda i: (1, i, 0)),
            pl.BlockSpec((_BN, D), lambda i: (i, 0)),
            pl.BlockSpec((_BN, D), lambda i: (i, 0)),
            pl.BlockSpec((_BN, 16), lambda i: (i, 0)),
            pl.BlockSpec((D, 4 * D), lambda i: (0, 0)),
            pl.BlockSpec((D, 4 * D), lambda i: (0, 0)),
            pl.BlockSpec((8, 4 * D), lambda i: (0, 0)),
        ],
        out_specs=pl.BlockSpec((_BN, D), lambda i: (i, 0)),
        out_shape=jax.ShapeDtypeStruct((N, D), jnp.float32),
    )(agg, agg, z, s, dinv16, wz, ws, bcat)


@jax.jit
def kernel(z, edge_index, s,
           Wi_reset, bi_reset, Ws_reset, bs_reset,
           Wi_update, bi_update, Ws_update, bs_update,
           Wi_cand, bi_cand, Ws_cand, bs_cand):
    ei = edge_index.astype(jnp.int32)
    npad = EPAD - E
    rowp = jnp.concatenate([ei[0], jnp.zeros((npad,), jnp.int32)]
                           ).reshape(IDXROWS, 128)
    colp = jnp.concatenate([ei[1], jnp.full((npad,), N, jnp.int32)]
                           ).reshape(IDXROWS, 128)

    degw = _deg_kernel(colp).reshape(NC, NPAD, 16)
    tz0, tz1, ts0, ts1, dinv16 = _scale_call(z, s, degw)
    agg = _agg_kernel(tz0, tz1, ts0, ts1, rowp, colp)
    agg = agg.reshape(NC, 2, NPAD, DH).transpose(0, 2, 1, 3).reshape(NC, NPAD, D)

    wz = jnp.concatenate(
        [Wi_reset, Wi_update, Wi_cand, jnp.zeros((D, D), jnp.float32)], axis=1)
    ws = jnp.concatenate(
        [Ws_reset, Ws_update, jnp.zeros((D, D), jnp.float32), Ws_cand], axis=1)
    bcat = jnp.concatenate([bi_reset + bs_reset, bi_update + bs_update,
                            bi_cand, bs_cand])
    bcat = jnp.tile(bcat[None, :], (8, 1))

    return _gru_call(agg, z, s, dinv16, wz, ws, bcat)


# speedup vs baseline: 21.4985x; 1.2234x over previous
import functools

import jax
import jax.numpy as jnp
from jax import lax
from jax.experimental import pallas as pl
from jax.experimental.pallas import tpu as pltpu
from jax.experimental.pallas import tpu_sc as plsc

N = 10000
E = 320000
D = 128
NC = 2
NS = 16
NPAD = 10240
ACC_ROWS = NPAD // NS
EPAD = 327680
IDXROWS = EPAD // 128
DEG_ROWS_PER_TILE = IDXROWS // (NC * NS)
KD = 8
ROWS_PER_TILE = IDXROWS // NS
K = 4

_MESH = plsc.VectorSubcoreMesh(core_axis_name="c", subcore_axis_name="s",
                               num_cores=NC, num_subcores=NS)


@functools.partial(
    pl.kernel,
    out_type=jax.ShapeDtypeStruct((NC * NPAD, 16), jnp.float32),
    mesh=_MESH,
    scratch_types=[
        pltpu.VMEM((KD, 128), jnp.int32),
        pltpu.VMEM((128, 16), jnp.float32),
        pltpu.VMEM((128, 16), jnp.float32),
        pltpu.VMEM_SHARED((NPAD, 16), jnp.float32),
    ],
)
def _deg_kernel(col_hbm, out_hbm, cidx_v, ones_v, zeros_v, acc_sh):
    c = lax.axis_index("c")
    s = lax.axis_index("s")

    def fill(i, _):
        ones_v[i] = jnp.ones((16,), jnp.float32)
        zeros_v[i] = jnp.zeros((16,), jnp.float32)
        return 0
    lax.fori_loop(0, 128, fill, 0)

    for q in range(ACC_ROWS // 128):
        pltpu.sync_copy(zeros_v, acc_sh.at[pl.ds(s * ACC_ROWS + q * 128, 128)])
    plsc.subcore_barrier()

    base = (c * NS + s) * DEG_ROWS_PER_TILE

    def chunk(it, _):
        pltpu.sync_copy(col_hbm.at[pl.ds(base + it * KD, KD)], cidx_v)
        for j in range(KD):
            pltpu.sync_copy(ones_v, acc_sh.at[cidx_v.at[j]], add=True)
        return 0
    lax.fori_loop(0, DEG_ROWS_PER_TILE // KD, chunk, 0)
    plsc.subcore_barrier()

    pltpu.sync_copy(acc_sh.at[pl.ds(s * ACC_ROWS, ACC_ROWS)],
                    out_hbm.at[pl.ds(c * NPAD + s * ACC_ROWS, ACC_ROWS)])


DH = D // 2


def _scale_body(z_ref, s_ref, degw_ref,
                tz0_ref, tz1_ref, ts0_ref, ts1_ref, dinv_ref):
    deg = degw_ref[0, :, 0:1] + degw_ref[1, :, 0:1] + 2.0
    dinv = lax.rsqrt(deg)
    tz = z_ref[...] * dinv
    ts = s_ref[...] * dinv
    tz0_ref[...] = tz[:, 0:DH]
    tz1_ref[...] = tz[:, DH:D]
    ts0_ref[...] = ts[:, 0:DH]
    ts1_ref[...] = ts[:, DH:D]
    dinv_ref[...] = jnp.broadcast_to(dinv, dinv_ref.shape)


_BN = 400


def _scale_call(z, s, degw):
    grid = N // _BN
    half = pl.BlockSpec((_BN, DH), lambda i: (i, 0))
    half_shape = jax.ShapeDtypeStruct((N, DH), jnp.float32)
    return pl.pallas_call(
        _scale_body,
        grid=(grid,),
        in_specs=[
            pl.BlockSpec((_BN, D), lambda i: (i, 0)),
            pl.BlockSpec((_BN, D), lambda i: (i, 0)),
            pl.BlockSpec((NC, _BN, 16), lambda i: (0, i, 0)),
        ],
        out_specs=[half, half, half, half,
                   pl.BlockSpec((_BN, 16), lambda i: (i, 0))],
        out_shape=[half_shape, half_shape, half_shape, half_shape,
                   jax.ShapeDtypeStruct((N, 16), jnp.float32)],
    )(z, s, degw)


@functools.partial(
    pl.kernel,
    out_type=jax.ShapeDtypeStruct((NC * 2 * NPAD, DH), jnp.float32),
    mesh=_MESH,
    compiler_params=pltpu.CompilerParams(use_tc_tiling_on_sc=False),
    scratch_types=[
        pltpu.VMEM((2 * K, 128), jnp.int32),
        pltpu.VMEM((2 * K, 128), jnp.int32),
        pltpu.VMEM((2 * K * 128, DH), jnp.float32),
        pltpu.VMEM((128, DH), jnp.float32),
        pltpu.VMEM_SHARED((NPAD, DH), jnp.float32),
        pltpu.SemaphoreType.DMA,
    ],
)
def _agg_kernel(tz0_hbm, tz1_hbm, ts0_hbm, ts1_hbm, row_hbm, col_hbm, out_hbm,
                ridx_v, cidx_v, gbuf_v, zeros_v, acc_sh, sem):
    c = lax.axis_index("c")
    s = lax.axis_index("s")

    def fill(i, _):
        for j in range(DH // 16):
            zeros_v[i, pl.ds(j * 16, 16)] = jnp.zeros((16,), jnp.float32)
        return 0
    lax.fori_loop(0, 128, fill, 0)

    base = s * ROWS_PER_TILE
    nchunk = ROWS_PER_TILE // K

    for h, (taba, tabb) in enumerate(((tz0_hbm, ts0_hbm), (tz1_hbm, ts1_hbm))):
        for q in range(ACC_ROWS // 128):
            pltpu.sync_copy(zeros_v,
                            acc_sh.at[pl.ds(s * ACC_ROWS + q * 128, 128)])
        plsc.subcore_barrier()

        def fire(it, p):
            pltpu.sync_copy(row_hbm.at[pl.ds(base + it * K, K)],
                            ridx_v.at[pl.ds(p * K, K)])
            pltpu.sync_copy(col_hbm.at[pl.ds(base + it * K, K)],
                            cidx_v.at[pl.ds(p * K, K)])

            @pl.when(c == 0)
            def _():
                for j in range(K):
                    pltpu.async_copy(taba.at[ridx_v.at[p * K + j]],
                                     gbuf_v.at[pl.ds((p * K + j) * 128, 128)],
                                     sem)

            @pl.when(c == 1)
            def _():
                for j in range(K):
                    pltpu.async_copy(tabb.at[ridx_v.at[p * K + j]],
                                     gbuf_v.at[pl.ds((p * K + j) * 128, 128)],
                                     sem)

        def drain(p):
            pltpu.make_async_copy(
                taba.at[pl.ds(0, K * 128)],
                gbuf_v.at[pl.ds(p * K * 128, K * 128)], sem).wait()

        def scatter(p):
            for j in range(K):
                pltpu.sync_copy(gbuf_v.at[pl.ds((p * K + j) * 128, 128)],
                                acc_sh.at[cidx_v.at[p * K + j]], add=True)

        fire(0, 0)

        def pair(t, _):
            it0 = t * 2
            drain(0)

            @pl.when(it0 + 1 < nchunk)
            def _():
                fire(it0 + 1, 1)
            scatter(0)
            drain(1)

            @pl.when(it0 + 2 < nchunk)
            def _():
                fire(it0 + 2, 0)
            scatter(1)
            return 0
        lax.fori_loop(0, nchunk // 2, pair, 0)
        plsc.subcore_barrier()

        pltpu.sync_copy(
            acc_sh.at[pl.ds(s * ACC_ROWS, ACC_ROWS)],
            out_hbm.at[pl.ds((c * 2 + h) * NPAD + s * ACC_ROWS, ACC_ROWS)])
        plsc.subcore_barrier()


def _gru_body(gz_ref, gs_ref, z_ref, s_ref, dinv_ref, wz_ref, ws_ref, b_ref,
              out_ref):
    d = dinv_ref[:, 0:1]
    two_d2 = 2.0 * d * d
    xz = d * gz_ref[0] + two_d2 * z_ref[...]
    xs = d * gs_ref[0] + two_d2 * s_ref[...]
    p = (jnp.dot(xz, wz_ref[...], preferred_element_type=jnp.float32,
                 precision=lax.Precision.HIGHEST)
         + jnp.dot(xs, ws_ref[...], preferred_element_type=jnp.float32,
                   precision=lax.Precision.HIGHEST)
         + b_ref[0:1, :])
    r = jax.nn.sigmoid(p[:, 0:D])
    u = jax.nn.sigmoid(p[:, D:2 * D])
    cand = jnp.tanh(p[:, 2 * D:3 * D] + r * p[:, 3 * D:4 * D])
    out_ref[...] = (1.0 - u) * cand + u * s_ref[...]


def _gru_call(agg, z, s, dinv16, wz, ws, bcat):
    grid = N // _BN
    return pl.pallas_call(
        _gru_body,
        grid=(grid,),
        in_specs=[
            pl.BlockSpec((1, _BN, D), lambda i: (0, i, 0)),
            pl.BlockSpec((1, _BN, D), lambda i: (1, i, 0)),
            pl.BlockSpec((_BN, D), lambda i: (i, 0)),
            pl.BlockSpec((_BN, D), lambda i: (i, 0)),
            pl.BlockSpec((_BN, 16), lambda i: (i, 0)),
            pl.BlockSpec((D, 4 * D), lambda i: (0, 0)),
            pl.BlockSpec((D, 4 * D), lambda i: (0, 0)),
            pl.BlockSpec((8, 4 * D), lambda i: (0, 0)),
        ],
        out_specs=pl.BlockSpec((_BN, D), lambda i: (i, 0)),
        out_shape=jax.ShapeDtypeStruct((N, D), jnp.float32),
    )(agg, agg, z, s, dinv16, wz, ws, bcat)


@jax.jit
def kernel(z, edge_index, s,
           Wi_reset, bi_reset, Ws_reset, bs_reset,
           Wi_update, bi_update, Ws_update, bs_update,
           Wi_cand, bi_cand, Ws_cand, bs_cand):
    ei = edge_index.astype(jnp.int32)
    npad = EPAD - E
    rowp = jnp.concatenate([ei[0], jnp.zeros((npad,), jnp.int32)]
                           ).reshape(IDXROWS, 128)
    colp = jnp.concatenate([ei[1], jnp.full((npad,), N, jnp.int32)]
                           ).reshape(IDXROWS, 128)

    degw = _deg_kernel(colp).reshape(NC, NPAD, 16)
    tz0, tz1, ts0, ts1, dinv16 = _scale_call(z, s, degw)
    agg = _agg_kernel(tz0, tz1, ts0, ts1, rowp, colp)
    agg = agg.reshape(NC, 2, NPAD, DH).transpose(0, 2, 1, 3).reshape(NC, NPAD, D)

    wz = jnp.concatenate(
        [Wi_reset, Wi_update, Wi_cand, jnp.zeros((D, D), jnp.float32)], axis=1)
    ws = jnp.concatenate(
        [Ws_reset, Ws_update, jnp.zeros((D, D), jnp.float32), Ws_cand], axis=1)
    bcat = jnp.concatenate([bi_reset + bs_reset, bi_update + bs_update,
                            bi_cand, bs_cand])
    bcat = jnp.tile(bcat[None, :], (8, 1))

    return _gru_call(agg, z, s, dinv16, wz, ws, bcat)
